# baseline (speedup 1.0000x reference)
import functools

import jax
import jax.numpy as jnp
from jax import lax
from jax.experimental import pallas as pl
from jax.experimental.pallas import tpu as pltpu
from jax.experimental.pallas import tpu_sc as plsc

N = 100000
E = 6400000
HID = 16
G = 64

NC = 2
NS = 16
NW = NC * NS
CH = 256
GRP = 8
EG = CH * GRP
NGROUPS = E // EG
BASE_GROUPS = NGROUPS // NW
EXTRA = NGROUPS % NW
ROW_SLC = 6256


def _h_body(x_ref, w1_ref, b1_ref, h_ref):
    h_ref[...] = jnp.tanh(
        jnp.dot(x_ref[...], w1_ref[...], preferred_element_type=jnp.float32)
        + b1_ref[...]
    )


def _compute_h(x, W1, b1):
    blk = 25000
    return pl.pallas_call(
        _h_body,
        grid=(N // blk,),
        in_specs=[
            pl.BlockSpec((blk, 3), lambda i: (i, 0)),
            pl.BlockSpec((3, HID), lambda i: (0, 0)),
            pl.BlockSpec((1, HID), lambda i: (0, 0)),
        ],
        out_specs=pl.BlockSpec((blk, HID), lambda i: (i, 0)),
        out_shape=jax.ShapeDtypeStruct((N, HID), jnp.float32),
    )(x, W1, b1.reshape(1, HID))


def _sc_body(row3, col3, h, ones_hbm, zeros_agg, zeros_deg,
             agg_out, deg_out,
             rowbuf, colbuf, rows0, rows1, ones_v, agg_sh, deg_sh,
             gsem0, gsem1, ssem0, ssem1, dsem):
    c = lax.axis_index("c")
    s = lax.axis_index("s")
    w = c * NS + s

    rstart = pl.multiple_of(jnp.where(s == NS - 1, N - ROW_SLC, s * ROW_SLC), 8)
    pltpu.sync_copy(
        zeros_agg.at[pl.ds(rstart, ROW_SLC)],
        agg_sh.at[pl.ds(rstart, ROW_SLC)],
    )

    @pl.when(s == 0)
    def _():
        pltpu.sync_copy(zeros_deg, deg_sh)

    pltpu.sync_copy(ones_hbm, ones_v)
    plsc.subcore_barrier()

    ngroups = jnp.where(w < EXTRA, BASE_GROUPS + 1, BASE_GROUPS)

    rows = [rows0, rows1]
    gsem = [gsem0, gsem1]
    ssem = [ssem0, ssem1]

    def group_body(j, carry):
        g = w + j * NW
        pltpu.sync_copy(row3.at[g], rowbuf)
        pltpu.sync_copy(col3.at[g], colbuf)
        gds = [None, None]
        sds = [None, None]
        dds = []
        gds[0] = pltpu.async_copy(h.at[colbuf.at[0]], rows[0], gsem[0])
        for i in range(GRP):
            p = i % 2
            q = 1 - p
            dds.append(pltpu.async_copy(
                ones_v, deg_sh.at[rowbuf.at[i]], dsem, add=True))
            if i + 1 < GRP:
                if sds[q] is not None:
                    sds[q].wait()
                gds[q] = pltpu.async_copy(h.at[colbuf.at[i + 1]],
                                          rows[q], gsem[q])
            gds[p].wait()
            sds[p] = pltpu.async_copy(rows[p], agg_sh.at[rowbuf.at[i]],
                                      ssem[p], add=True)
        sds[0].wait()
        sds[1].wait()
        for d in dds:
            d.wait()
        return carry

    lax.fori_loop(0, ngroups, group_body, 0)
    plsc.subcore_barrier()

    pltpu.sync_copy(
        agg_sh.at[pl.ds(rstart, ROW_SLC)],
        agg_out.at[c, pl.ds(rstart, ROW_SLC)],
    )

    @pl.when(s == 0)
    def _():
        pltpu.sync_copy(deg_sh, deg_out.at[c])


def _sc_aggregate(row, col, h):
    row3 = row.reshape(NGROUPS, GRP, CH)
    col3 = col.reshape(NGROUPS, GRP, CH)
    mesh = plsc.VectorSubcoreMesh(core_axis_name="c", subcore_axis_name="s")
    kern = pl.kernel(
        _sc_body,
        mesh=mesh,
        compiler_params=pltpu.CompilerParams(use_tc_tiling_on_sc=False),
        out_type=(
            jax.ShapeDtypeStruct((NC, N, HID), jnp.float32),
            jax.ShapeDtypeStruct((NC, N), jnp.float32),
        ),
        scratch_types=[
            pltpu.VMEM((GRP, CH), jnp.int32),
            pltpu.VMEM((GRP, CH), jnp.int32),
            pltpu.VMEM((CH, HID), jnp.float32),
            pltpu.VMEM((CH, HID), jnp.float32),
            pltpu.VMEM((CH,), jnp.float32),
            pltpu.VMEM_SHARED((N, HID), jnp.float32),
            pltpu.VMEM_SHARED((N,), jnp.float32),
            pltpu.SemaphoreType.DMA,
            pltpu.SemaphoreType.DMA,
            pltpu.SemaphoreType.DMA,
            pltpu.SemaphoreType.DMA,
            pltpu.SemaphoreType.DMA,
        ],
    )
    ones = jnp.ones((CH,), jnp.float32)
    zagg = jnp.zeros((N, HID), jnp.float32)
    zdeg = jnp.zeros((N,), jnp.float32)
    return kern(row3, col3, h, ones, zagg, zdeg)


B3 = 512
NB3 = (N + B3 - 1) // B3


def _pool_body(a0_ref, a1_ref, d0_ref, d1_ref, batch_ref,
               w2_ref, b2_ref, w3_ref, b3_ref, out_ref,
               mean_acc, max_acc, cnt_acc):
    i = pl.program_id(0)

    @pl.when(i == 0)
    def _():
        mean_acc[...] = jnp.zeros((HID, G), jnp.float32)
        max_acc[...] = jnp.full((HID, G), -jnp.inf, jnp.float32)
        cnt_acc[...] = jnp.zeros((1, G), jnp.float32)

    agg = a0_ref[...] + a1_ref[...]
    deg = jnp.maximum(d0_ref[...] + d1_ref[...], 1.0)
    z = (
        jnp.dot(agg / deg[:, None], w2_ref[...],
                preferred_element_type=jnp.float32)
        + b2_ref[...]
    )
    nid = i * B3 + lax.broadcasted_iota(jnp.int32, (B3, 1), 0)
    valid = nid < N
    h2 = jnp.where(valid, jnp.tanh(z), 0.0)

    gids = lax.broadcasted_iota(jnp.int32, (B3, G), 1)
    p = (batch_ref[...][:, None] == gids) & valid
    pf = p.astype(jnp.float32)

    mean_acc[...] += lax.dot_general(
        h2, pf, (((0,), (0,)), ((), ())),
        preferred_element_type=jnp.float32)
    cnt_acc[...] += jnp.sum(pf, axis=0, keepdims=True)

    madd = jnp.where(p, 0.0, -jnp.inf)
    mxs = jnp.concatenate(
        [jnp.max(madd + h2[:, d:d + 1], axis=0, keepdims=True)
         for d in range(HID)], axis=0)
    max_acc[...] = jnp.maximum(max_acc[...], mxs)

    @pl.when(i == NB3 - 1)
    def _():
        hm = mean_acc[...] / jnp.maximum(cnt_acc[...], 1.0)
        hx = max_acc[...]
        w3a = w3_ref[0:HID, :]
        w3b = w3_ref[HID:2 * HID, :]
        r = lax.dot_general(w3a, hm, (((0,), (0,)), ((), ())),
                            preferred_element_type=jnp.float32)
        r += lax.dot_general(w3b, hx, (((0,), (0,)), ((), ())),
                             preferred_element_type=jnp.float32)
        out_ref[...] = jax.nn.sigmoid(r + b3_ref[...])


def _pool(aggp, degp, batch, W2, b2, W3, b3):
    out = pl.pallas_call(
        _pool_body,
        grid=(NB3,),
        in_specs=[
            pl.BlockSpec((B3, HID), lambda i: (i, 0)),
            pl.BlockSpec((B3, HID), lambda i: (i, 0)),
            pl.BlockSpec((B3,), lambda i: (i,)),
            pl.BlockSpec((B3,), lambda i: (i,)),
            pl.BlockSpec((B3,), lambda i: (i,)),
            pl.BlockSpec((HID, HID), lambda i: (0, 0)),
            pl.BlockSpec((1, HID), lambda i: (0, 0)),
            pl.BlockSpec((2 * HID, 1), lambda i: (0, 0)),
            pl.BlockSpec((1, 1), lambda i: (0, 0)),
        ],
        out_specs=pl.BlockSpec((1, G), lambda i: (0, 0)),
        out_shape=jax.ShapeDtypeStruct((1, G), jnp.float32),
        scratch_shapes=[
            pltpu.VMEM((HID, G), jnp.float32),
            pltpu.VMEM((HID, G), jnp.float32),
            pltpu.VMEM((1, G), jnp.float32),
        ],
    )(aggp[0], aggp[1], degp[0], degp[1], batch,
      W2, b2.reshape(1, HID), W3, b3.reshape(1, 1))
    return out.reshape(G)


def kernel(x, edge_index, batch, W1, b1, W2, b2, W3, b3):
    h = _compute_h(x, W1, b1)
    aggp, degp = _sc_aggregate(edge_index[0], edge_index[1], h)
    return _pool(aggp, degp, batch, W2, b2, W3, b3)


# speedup vs baseline: 1.0930x; 1.0930x over previous
import functools

import jax
import jax.numpy as jnp
from jax import lax
from jax.experimental import pallas as pl
from jax.experimental.pallas import tpu as pltpu
from jax.experimental.pallas import tpu_sc as plsc

N = 100000
E = 6400000
HID = 16
G = 64

NC = 2
NS = 16
NW = NC * NS
CH = 256
GRP = 8
EG = CH * GRP
NGROUPS = E // EG
BASE_GROUPS = NGROUPS // NW
EXTRA = NGROUPS % NW
ROW_SLC = 6256


def _h_body(x_ref, w1_ref, b1_ref, h_ref):
    h_ref[...] = jnp.tanh(
        jnp.dot(x_ref[...], w1_ref[...], preferred_element_type=jnp.float32)
        + b1_ref[...]
    )


def _compute_h(x, W1, b1):
    blk = 25000
    return pl.pallas_call(
        _h_body,
        grid=(N // blk,),
        in_specs=[
            pl.BlockSpec((blk, 3), lambda i: (i, 0)),
            pl.BlockSpec((3, HID), lambda i: (0, 0)),
            pl.BlockSpec((1, HID), lambda i: (0, 0)),
        ],
        out_specs=pl.BlockSpec((blk, HID), lambda i: (i, 0)),
        out_shape=jax.ShapeDtypeStruct((N, HID), jnp.float32),
    )(x, W1, b1.reshape(1, HID))


def _sc_body(row3, col3, h, ones_hbm, zeros_agg, zeros_deg,
             agg_out, deg_out,
             rowbuf, colbuf, rows0, rows1, ones_v, agg_sh, deg_sh,
             gsem0, gsem1, ssem0, ssem1, dsem, isem):
    c = lax.axis_index("c")
    s = lax.axis_index("s")
    w = c * NS + s

    rstart = pl.multiple_of(jnp.where(s == NS - 1, N - ROW_SLC, s * ROW_SLC), 8)
    pltpu.sync_copy(
        zeros_agg.at[pl.ds(rstart, ROW_SLC)],
        agg_sh.at[pl.ds(rstart, ROW_SLC)],
    )

    @pl.when(s == 0)
    def _():
        pltpu.sync_copy(zeros_deg, deg_sh)

    pltpu.sync_copy(ones_hbm, ones_v)
    plsc.subcore_barrier()

    ngroups = jnp.where(w < EXTRA, BASE_GROUPS + 1, BASE_GROUPS)

    rows = [rows0, rows1]
    gsem = [gsem0, gsem1]
    ssem = [ssem0, ssem1]

    pltpu.sync_copy(row3.at[w], rowbuf.at[0])
    pltpu.sync_copy(col3.at[w], colbuf.at[0])

    def group_body(j, carry):
        p2 = j % 2
        q2 = 1 - p2
        gnext = jnp.minimum(w + (j + 1) * NW, NGROUPS - 1)
        ird = pltpu.async_copy(row3.at[gnext], rowbuf.at[q2], isem)
        icd = pltpu.async_copy(col3.at[gnext], colbuf.at[q2], isem)
        gds = [None, None]
        sds = [None, None]
        dds = []
        gds[0] = pltpu.async_copy(h.at[colbuf.at[p2, 0]], rows[0], gsem[0])
        for i in range(GRP):
            p = i % 2
            q = 1 - p
            dds.append(pltpu.async_copy(
                ones_v, deg_sh.at[rowbuf.at[p2, i]], dsem, add=True))
            if i + 1 < GRP:
                if sds[q] is not None:
                    sds[q].wait()
                gds[q] = pltpu.async_copy(h.at[colbuf.at[p2, i + 1]],
                                          rows[q], gsem[q])
            gds[p].wait()
            sds[p] = pltpu.async_copy(rows[p], agg_sh.at[rowbuf.at[p2, i]],
                                      ssem[p], add=True)
        sds[0].wait()
        sds[1].wait()
        for d in dds:
            d.wait()
        ird.wait()
        icd.wait()
        return carry

    lax.fori_loop(0, ngroups, group_body, 0)
    plsc.subcore_barrier()

    pltpu.sync_copy(
        agg_sh.at[pl.ds(rstart, ROW_SLC)],
        agg_out.at[c, pl.ds(rstart, ROW_SLC)],
    )

    @pl.when(s == 0)
    def _():
        pltpu.sync_copy(deg_sh, deg_out.at[c])


def _sc_aggregate(row, col, h):
    row3 = row.reshape(NGROUPS, GRP, CH)
    col3 = col.reshape(NGROUPS, GRP, CH)
    mesh = plsc.VectorSubcoreMesh(core_axis_name="c", subcore_axis_name="s")
    kern = pl.kernel(
        _sc_body,
        mesh=mesh,
        compiler_params=pltpu.CompilerParams(use_tc_tiling_on_sc=False),
        out_type=(
            jax.ShapeDtypeStruct((NC, N, HID), jnp.float32),
            jax.ShapeDtypeStruct((NC, N), jnp.float32),
        ),
        scratch_types=[
            pltpu.VMEM((2, GRP, CH), jnp.int32),
            pltpu.VMEM((2, GRP, CH), jnp.int32),
            pltpu.VMEM((CH, HID), jnp.float32),
            pltpu.VMEM((CH, HID), jnp.float32),
            pltpu.VMEM((CH,), jnp.float32),
            pltpu.VMEM_SHARED((N, HID), jnp.float32),
            pltpu.VMEM_SHARED((N,), jnp.float32),
            pltpu.SemaphoreType.DMA,
            pltpu.SemaphoreType.DMA,
            pltpu.SemaphoreType.DMA,
            pltpu.SemaphoreType.DMA,
            pltpu.SemaphoreType.DMA,
            pltpu.SemaphoreType.DMA,
        ],
    )
    ones = jnp.ones((CH,), jnp.float32)
    zagg = jnp.zeros((N, HID), jnp.float32)
    zdeg = jnp.zeros((N,), jnp.float32)
    return kern(row3, col3, h, ones, zagg, zdeg)


B3 = 512
NB3 = (N + B3 - 1) // B3


def _pool_body(a0_ref, a1_ref, d0_ref, d1_ref, batch_ref,
               w2_ref, b2_ref, w3_ref, b3_ref, out_ref,
               mean_acc, max_acc, cnt_acc):
    i = pl.program_id(0)

    @pl.when(i == 0)
    def _():
        mean_acc[...] = jnp.zeros((HID, G), jnp.float32)
        max_acc[...] = jnp.full((HID, G), -jnp.inf, jnp.float32)
        cnt_acc[...] = jnp.zeros((1, G), jnp.float32)

    agg = a0_ref[...] + a1_ref[...]
    deg = jnp.maximum(d0_ref[...] + d1_ref[...], 1.0)
    z = (
        jnp.dot(agg / deg[:, None], w2_ref[...],
                preferred_element_type=jnp.float32)
        + b2_ref[...]
    )
    nid = i * B3 + lax.broadcasted_iota(jnp.int32, (B3, 1), 0)
    valid = nid < N
    h2 = jnp.where(valid, jnp.tanh(z), 0.0)

    gids = lax.broadcasted_iota(jnp.int32, (B3, G), 1)
    p = (batch_ref[...][:, None] == gids) & valid
    pf = p.astype(jnp.float32)

    mean_acc[...] += lax.dot_general(
        h2, pf, (((0,), (0,)), ((), ())),
        preferred_element_type=jnp.float32)
    cnt_acc[...] += jnp.sum(pf, axis=0, keepdims=True)

    madd = jnp.where(p, 0.0, -jnp.inf)
    mxs = jnp.concatenate(
        [jnp.max(madd + h2[:, d:d + 1], axis=0, keepdims=True)
         for d in range(HID)], axis=0)
    max_acc[...] = jnp.maximum(max_acc[...], mxs)

    @pl.when(i == NB3 - 1)
    def _():
        hm = mean_acc[...] / jnp.maximum(cnt_acc[...], 1.0)
        hx = max_acc[...]
        w3a = w3_ref[0:HID, :]
        w3b = w3_ref[HID:2 * HID, :]
        r = lax.dot_general(w3a, hm, (((0,), (0,)), ((), ())),
                            preferred_element_type=jnp.float32)
        r += lax.dot_general(w3b, hx, (((0,), (0,)), ((), ())),
                             preferred_element_type=jnp.float32)
        out_ref[...] = jax.nn.sigmoid(r + b3_ref[...])


def _pool(aggp, degp, batch, W2, b2, W3, b3):
    out = pl.pallas_call(
        _pool_body,
        grid=(NB3,),
        in_specs=[
            pl.BlockSpec((B3, HID), lambda i: (i, 0)),
            pl.BlockSpec((B3, HID), lambda i: (i, 0)),
            pl.BlockSpec((B3,), lambda i: (i,)),
            pl.BlockSpec((B3,), lambda i: (i,)),
            pl.BlockSpec((B3,), lambda i: (i,)),
            pl.BlockSpec((HID, HID), lambda i: (0, 0)),
            pl.BlockSpec((1, HID), lambda i: (0, 0)),
            pl.BlockSpec((2 * HID, 1), lambda i: (0, 0)),
            pl.BlockSpec((1, 1), lambda i: (0, 0)),
        ],
        out_specs=pl.BlockSpec((1, G), lambda i: (0, 0)),
        out_shape=jax.ShapeDtypeStruct((1, G), jnp.float32),
        scratch_shapes=[
            pltpu.VMEM((HID, G), jnp.float32),
            pltpu.VMEM((HID, G), jnp.float32),
            pltpu.VMEM((1, G), jnp.float32),
        ],
    )(aggp[0], aggp[1], degp[0], degp[1], batch,
      W2, b2.reshape(1, HID), W3, b3.reshape(1, 1))
    return out.reshape(G)


def kernel(x, edge_index, batch, W1, b1, W2, b2, W3, b3):
    h = _compute_h(x, W1, b1)
    aggp, degp = _sc_aggregate(edge_index[0], edge_index[1], h)
    return _pool(aggp, degp, batch, W2, b2, W3, b3)


# speedup vs baseline: 1.0939x; 1.0008x over previous
import functools

import jax
import jax.numpy as jnp
from jax import lax
from jax.experimental import pallas as pl
from jax.experimental.pallas import tpu as pltpu
from jax.experimental.pallas import tpu_sc as plsc

N = 100000
E = 6400000
HID = 16
G = 64

NC = 2
NS = 16
NW = NC * NS
CH = 256
GRP = 8
EG = CH * GRP
NGROUPS = E // EG
BASE_GROUPS = NGROUPS // NW
EXTRA = NGROUPS % NW
ROW_SLC = 6256


def _h_body(x_ref, w1_ref, b1_ref, h_ref):
    h_ref[...] = jnp.tanh(
        jnp.dot(x_ref[...], w1_ref[...], preferred_element_type=jnp.float32)
        + b1_ref[...]
    )


def _compute_h(x, W1, b1):
    blk = 25000
    return pl.pallas_call(
        _h_body,
        grid=(N // blk,),
        in_specs=[
            pl.BlockSpec((blk, 3), lambda i: (i, 0)),
            pl.BlockSpec((3, HID), lambda i: (0, 0)),
            pl.BlockSpec((1, HID), lambda i: (0, 0)),
        ],
        out_specs=pl.BlockSpec((blk, HID), lambda i: (i, 0)),
        out_shape=jax.ShapeDtypeStruct((N, HID), jnp.float32),
    )(x, W1, b1.reshape(1, HID))


def _sc_body(row3, col3, h, ones_hbm, zeros_agg, zeros_deg,
             agg_out, deg_out,
             rowbuf, colbuf, rows0, rows1, ones_v, agg_sh, deg_sh,
             gsem0, gsem1, ssem0, ssem1, dsem, isem):
    c = lax.axis_index("c")
    s = lax.axis_index("s")
    w = c * NS + s

    rstart = pl.multiple_of(jnp.where(s == NS - 1, N - ROW_SLC, s * ROW_SLC), 8)
    pltpu.sync_copy(
        zeros_agg.at[pl.ds(rstart, ROW_SLC)],
        agg_sh.at[pl.ds(rstart, ROW_SLC)],
    )

    @pl.when(s == 0)
    def _():
        pltpu.sync_copy(zeros_deg, deg_sh)

    pltpu.sync_copy(ones_hbm, ones_v)
    plsc.subcore_barrier()

    ngroups = jnp.where(w < EXTRA, BASE_GROUPS + 1, BASE_GROUPS)

    rows = [rows0, rows1]
    gsem = [gsem0, gsem1]
    ssem = [ssem0, ssem1]

    pltpu.sync_copy(row3.at[w], rowbuf.at[0])
    pltpu.sync_copy(col3.at[w], colbuf.at[0])

    def drain_tail(p2):
        pltpu.make_async_copy(
            rows[0], agg_sh.at[rowbuf.at[p2, 0]], ssem[0]).wait()
        pltpu.make_async_copy(
            rows[1], agg_sh.at[rowbuf.at[p2, 1]], ssem[1]).wait()
        for i in range(GRP):
            pltpu.make_async_copy(
                ones_v, deg_sh.at[rowbuf.at[p2, i]], dsem).wait()
        pltpu.make_async_copy(row3.at[w], rowbuf.at[p2], isem).wait()
        pltpu.make_async_copy(col3.at[w], colbuf.at[p2], isem).wait()

    def group_body(j, carry):
        p2 = j % 2
        q2 = 1 - p2

        @pl.when(j > 0)
        def _():
            drain_tail(q2)

        gnext = jnp.minimum(w + (j + 1) * NW, NGROUPS - 1)
        pltpu.async_copy(row3.at[gnext], rowbuf.at[q2], isem)
        pltpu.async_copy(col3.at[gnext], colbuf.at[q2], isem)
        gds = [None, None]
        sds = [None, None]
        gds[0] = pltpu.async_copy(h.at[colbuf.at[p2, 0]], rows[0], gsem[0])
        for i in range(GRP):
            p = i % 2
            q = 1 - p
            pltpu.async_copy(ones_v, deg_sh.at[rowbuf.at[p2, i]], dsem,
                             add=True)
            if i + 1 < GRP:
                if sds[q] is not None:
                    sds[q].wait()
                gds[q] = pltpu.async_copy(h.at[colbuf.at[p2, i + 1]],
                                          rows[q], gsem[q])
            gds[p].wait()
            sds[p] = pltpu.async_copy(rows[p], agg_sh.at[rowbuf.at[p2, i]],
                                      ssem[p], add=True)
        return carry

    lax.fori_loop(0, ngroups, group_body, 0)
    drain_tail((ngroups - 1) % 2)
    plsc.subcore_barrier()

    pltpu.sync_copy(
        agg_sh.at[pl.ds(rstart, ROW_SLC)],
        agg_out.at[c, pl.ds(rstart, ROW_SLC)],
    )

    @pl.when(s == 0)
    def _():
        pltpu.sync_copy(deg_sh, deg_out.at[c])


def _sc_aggregate(row, col, h):
    row3 = row.reshape(NGROUPS, GRP, CH)
    col3 = col.reshape(NGROUPS, GRP, CH)
    mesh = plsc.VectorSubcoreMesh(core_axis_name="c", subcore_axis_name="s")
    kern = pl.kernel(
        _sc_body,
        mesh=mesh,
        compiler_params=pltpu.CompilerParams(use_tc_tiling_on_sc=False),
        out_type=(
            jax.ShapeDtypeStruct((NC, N, HID), jnp.float32),
            jax.ShapeDtypeStruct((NC, N), jnp.float32),
        ),
        scratch_types=[
            pltpu.VMEM((2, GRP, CH), jnp.int32),
            pltpu.VMEM((2, GRP, CH), jnp.int32),
            pltpu.VMEM((CH, HID), jnp.float32),
            pltpu.VMEM((CH, HID), jnp.float32),
            pltpu.VMEM((CH,), jnp.float32),
            pltpu.VMEM_SHARED((N, HID), jnp.float32),
            pltpu.VMEM_SHARED((N,), jnp.float32),
            pltpu.SemaphoreType.DMA,
            pltpu.SemaphoreType.DMA,
            pltpu.SemaphoreType.DMA,
            pltpu.SemaphoreType.DMA,
            pltpu.SemaphoreType.DMA,
            pltpu.SemaphoreType.DMA,
        ],
    )
    ones = jnp.ones((CH,), jnp.float32)
    zagg = jnp.zeros((N, HID), jnp.float32)
    zdeg = jnp.zeros((N,), jnp.float32)
    return kern(row3, col3, h, ones, zagg, zdeg)


B3 = 512
NB3 = (N + B3 - 1) // B3


def _pool_body(a0_ref, a1_ref, d0_ref, d1_ref, batch_ref,
               w2_ref, b2_ref, w3_ref, b3_ref, out_ref,
               mean_acc, max_acc, cnt_acc):
    i = pl.program_id(0)

    @pl.when(i == 0)
    def _():
        mean_acc[...] = jnp.zeros((HID, G), jnp.float32)
        max_acc[...] = jnp.full((HID, G), -jnp.inf, jnp.float32)
        cnt_acc[...] = jnp.zeros((1, G), jnp.float32)

    agg = a0_ref[...] + a1_ref[...]
    deg = jnp.maximum(d0_ref[...] + d1_ref[...], 1.0)
    z = (
        jnp.dot(agg / deg[:, None], w2_ref[...],
                preferred_element_type=jnp.float32)
        + b2_ref[...]
    )
    nid = i * B3 + lax.broadcasted_iota(jnp.int32, (B3, 1), 0)
    valid = nid < N
    h2 = jnp.where(valid, jnp.tanh(z), 0.0)

    gids = lax.broadcasted_iota(jnp.int32, (B3, G), 1)
    p = (batch_ref[...][:, None] == gids) & valid
    pf = p.astype(jnp.float32)

    mean_acc[...] += lax.dot_general(
        h2, pf, (((0,), (0,)), ((), ())),
        preferred_element_type=jnp.float32)
    cnt_acc[...] += jnp.sum(pf, axis=0, keepdims=True)

    madd = jnp.where(p, 0.0, -jnp.inf)
    mxs = jnp.concatenate(
        [jnp.max(madd + h2[:, d:d + 1], axis=0, keepdims=True)
         for d in range(HID)], axis=0)
    max_acc[...] = jnp.maximum(max_acc[...], mxs)

    @pl.when(i == NB3 - 1)
    def _():
        hm = mean_acc[...] / jnp.maximum(cnt_acc[...], 1.0)
        hx = max_acc[...]
        w3a = w3_ref[0:HID, :]
        w3b = w3_ref[HID:2 * HID, :]
        r = lax.dot_general(w3a, hm, (((0,), (0,)), ((), ())),
                            preferred_element_type=jnp.float32)
        r += lax.dot_general(w3b, hx, (((0,), (0,)), ((), ())),
                             preferred_element_type=jnp.float32)
        out_ref[...] = jax.nn.sigmoid(r + b3_ref[...])


def _pool(aggp, degp, batch, W2, b2, W3, b3):
    out = pl.pallas_call(
        _pool_body,
        grid=(NB3,),
        in_specs=[
            pl.BlockSpec((B3, HID), lambda i: (i, 0)),
            pl.BlockSpec((B3, HID), lambda i: (i, 0)),
            pl.BlockSpec((B3,), lambda i: (i,)),
            pl.BlockSpec((B3,), lambda i: (i,)),
            pl.BlockSpec((B3,), lambda i: (i,)),
            pl.BlockSpec((HID, HID), lambda i: (0, 0)),
            pl.BlockSpec((1, HID), lambda i: (0, 0)),
            pl.BlockSpec((2 * HID, 1), lambda i: (0, 0)),
            pl.BlockSpec((1, 1), lambda i: (0, 0)),
        ],
        out_specs=pl.BlockSpec((1, G), lambda i: (0, 0)),
        out_shape=jax.ShapeDtypeStruct((1, G), jnp.float32),
        scratch_shapes=[
            pltpu.VMEM((HID, G), jnp.float32),
            pltpu.VMEM((HID, G), jnp.float32),
            pltpu.VMEM((1, G), jnp.float32),
        ],
    )(aggp[0], aggp[1], degp[0], degp[1], batch,
      W2, b2.reshape(1, HID), W3, b3.reshape(1, 1))
    return out.reshape(G)


def kernel(x, edge_index, batch, W1, b1, W2, b2, W3, b3):
    h = _compute_h(x, W1, b1)
    aggp, degp = _sc_aggregate(edge_index[0], edge_index[1], h)
    return _pool(aggp, degp, batch, W2, b2, W3, b3)


# speedup vs baseline: 1.1994x; 1.0965x over previous
import functools

import jax
import jax.numpy as jnp
from jax import lax
from jax.experimental import pallas as pl
from jax.experimental.pallas import tpu as pltpu
from jax.experimental.pallas import tpu_sc as plsc

N = 100000
E = 6400000
HID = 16
G = 64

NC = 2
NS = 16
NW = NC * NS
CH = 256
GRP = 8
EG = CH * GRP
NGROUPS = E // EG
BASE_GROUPS = NGROUPS // NW
EXTRA = NGROUPS % NW
ROW_SLC = 6256


def _h_body(x_ref, w1_ref, b1_ref, h_ref):
    h_ref[...] = jnp.tanh(
        jnp.dot(x_ref[...], w1_ref[...], preferred_element_type=jnp.float32)
        + b1_ref[...]
    )


def _compute_h(x, W1, b1):
    blk = 25000
    return pl.pallas_call(
        _h_body,
        grid=(N // blk,),
        in_specs=[
            pl.BlockSpec((blk, 3), lambda i: (i, 0)),
            pl.BlockSpec((3, HID), lambda i: (0, 0)),
            pl.BlockSpec((1, HID), lambda i: (0, 0)),
        ],
        out_specs=pl.BlockSpec((blk, HID), lambda i: (i, 0)),
        out_shape=jax.ShapeDtypeStruct((N, HID), jnp.float32),
    )(x, W1, b1.reshape(1, HID))


def _sc_body(row3, col3, h, ones_hbm, zeros_agg, zeros_deg,
             agg_out, deg_out,
             rowbuf, colbuf, rows0, rows1, rows2, ones_v, agg_sh, deg_sh,
             gsem0, gsem1, gsem2, ssem0, ssem1, ssem2, dsem, isem):
    c = lax.axis_index("c")
    s = lax.axis_index("s")
    w = c * NS + s

    rstart = pl.multiple_of(jnp.where(s == NS - 1, N - ROW_SLC, s * ROW_SLC), 8)
    pltpu.sync_copy(
        zeros_agg.at[pl.ds(rstart, ROW_SLC)],
        agg_sh.at[pl.ds(rstart, ROW_SLC)],
    )

    @pl.when(s == 0)
    def _():
        pltpu.sync_copy(zeros_deg, deg_sh)

    pltpu.sync_copy(ones_hbm, ones_v)
    plsc.subcore_barrier()

    ngroups = jnp.where(w < EXTRA, BASE_GROUPS + 1, BASE_GROUPS)

    rows = [rows0, rows1, rows2]
    gsem = [gsem0, gsem1, gsem2]
    ssem = [ssem0, ssem1, ssem2]
    tail = [(GRP - 3 + k) % 3 for k in range(3)]

    pltpu.sync_copy(row3.at[w], rowbuf.at[0])
    pltpu.sync_copy(col3.at[w], colbuf.at[0])

    def drain_tail(p2):
        for b in tail:
            pltpu.make_async_copy(
                rows[b], agg_sh.at[rowbuf.at[p2, 0]], ssem[b]).wait()
        for i in range(GRP):
            pltpu.make_async_copy(
                ones_v, deg_sh.at[rowbuf.at[p2, i]], dsem).wait()
        pltpu.make_async_copy(row3.at[w], rowbuf.at[p2], isem).wait()
        pltpu.make_async_copy(col3.at[w], colbuf.at[p2], isem).wait()

    def group_body(j, carry):
        p2 = j % 2
        q2 = 1 - p2

        @pl.when(j > 0)
        def _():
            drain_tail(q2)

        gnext = jnp.minimum(w + (j + 1) * NW, NGROUPS - 1)
        pltpu.async_copy(row3.at[gnext], rowbuf.at[q2], isem)
        pltpu.async_copy(col3.at[gnext], colbuf.at[q2], isem)
        gds = [None] * GRP
        sds = [None] * GRP
        gds[0] = pltpu.async_copy(h.at[colbuf.at[p2, 0]], rows[0], gsem[0])
        gds[1] = pltpu.async_copy(h.at[colbuf.at[p2, 1]], rows[1], gsem[1])
        for i in range(GRP):
            b = i % 3
            pltpu.async_copy(ones_v, deg_sh.at[rowbuf.at[p2, i]], dsem,
                             add=True)
            if i + 2 < GRP:
                if i >= 1:
                    sds[i - 1].wait()
                nb = (i + 2) % 3
                gds[i + 2] = pltpu.async_copy(h.at[colbuf.at[p2, i + 2]],
                                              rows[nb], gsem[nb])
            gds[i].wait()
            sds[i] = pltpu.async_copy(rows[b], agg_sh.at[rowbuf.at[p2, i]],
                                      ssem[b], add=True)
        return carry

    lax.fori_loop(0, ngroups, group_body, 0)
    drain_tail((ngroups - 1) % 2)
    plsc.subcore_barrier()

    pltpu.sync_copy(
        agg_sh.at[pl.ds(rstart, ROW_SLC)],
        agg_out.at[c, pl.ds(rstart, ROW_SLC)],
    )

    @pl.when(s == 0)
    def _():
        pltpu.sync_copy(deg_sh, deg_out.at[c])


def _sc_aggregate(row, col, h):
    row3 = row.reshape(NGROUPS, GRP, CH)
    col3 = col.reshape(NGROUPS, GRP, CH)
    mesh = plsc.VectorSubcoreMesh(core_axis_name="c", subcore_axis_name="s")
    kern = pl.kernel(
        _sc_body,
        mesh=mesh,
        compiler_params=pltpu.CompilerParams(use_tc_tiling_on_sc=False),
        out_type=(
            jax.ShapeDtypeStruct((NC, N, HID), jnp.float32),
            jax.ShapeDtypeStruct((NC, N), jnp.float32),
        ),
        scratch_types=[
            pltpu.VMEM((2, GRP, CH), jnp.int32),
            pltpu.VMEM((2, GRP, CH), jnp.int32),
            pltpu.VMEM((CH, HID), jnp.float32),
            pltpu.VMEM((CH, HID), jnp.float32),
            pltpu.VMEM((CH, HID), jnp.float32),
            pltpu.VMEM((CH,), jnp.float32),
            pltpu.VMEM_SHARED((N, HID), jnp.float32),
            pltpu.VMEM_SHARED((N,), jnp.float32),
            pltpu.SemaphoreType.DMA,
            pltpu.SemaphoreType.DMA,
            pltpu.SemaphoreType.DMA,
            pltpu.SemaphoreType.DMA,
            pltpu.SemaphoreType.DMA,
            pltpu.SemaphoreType.DMA,
            pltpu.SemaphoreType.DMA,
            pltpu.SemaphoreType.DMA,
        ],
    )
    ones = jnp.ones((CH,), jnp.float32)
    zagg = jnp.zeros((N, HID), jnp.float32)
    zdeg = jnp.zeros((N,), jnp.float32)
    return kern(row3, col3, h, ones, zagg, zdeg)


B3 = 512
NB3 = (N + B3 - 1) // B3


def _pool_body(a0_ref, a1_ref, d0_ref, d1_ref, batch_ref,
               w2_ref, b2_ref, w3_ref, b3_ref, out_ref,
               mean_acc, max_acc, cnt_acc):
    i = pl.program_id(0)

    @pl.when(i == 0)
    def _():
        mean_acc[...] = jnp.zeros((HID, G), jnp.float32)
        max_acc[...] = jnp.full((HID, G), -jnp.inf, jnp.float32)
        cnt_acc[...] = jnp.zeros((1, G), jnp.float32)

    agg = a0_ref[...] + a1_ref[...]
    deg = jnp.maximum(d0_ref[...] + d1_ref[...], 1.0)
    z = (
        jnp.dot(agg / deg[:, None], w2_ref[...],
                preferred_element_type=jnp.float32)
        + b2_ref[...]
    )
    nid = i * B3 + lax.broadcasted_iota(jnp.int32, (B3, 1), 0)
    valid = nid < N
    h2 = jnp.where(valid, jnp.tanh(z), 0.0)

    gids = lax.broadcasted_iota(jnp.int32, (B3, G), 1)
    p = (batch_ref[...][:, None] == gids) & valid
    pf = p.astype(jnp.float32)

    mean_acc[...] += lax.dot_general(
        h2, pf, (((0,), (0,)), ((), ())),
        preferred_element_type=jnp.float32)
    cnt_acc[...] += jnp.sum(pf, axis=0, keepdims=True)

    madd = jnp.where(p, 0.0, -jnp.inf)
    mxs = jnp.concatenate(
        [jnp.max(madd + h2[:, d:d + 1], axis=0, keepdims=True)
         for d in range(HID)], axis=0)
    max_acc[...] = jnp.maximum(max_acc[...], mxs)

    @pl.when(i == NB3 - 1)
    def _():
        hm = mean_acc[...] / jnp.maximum(cnt_acc[...], 1.0)
        hx = max_acc[...]
        w3a = w3_ref[0:HID, :]
        w3b = w3_ref[HID:2 * HID, :]
        r = lax.dot_general(w3a, hm, (((0,), (0,)), ((), ())),
                            preferred_element_type=jnp.float32)
        r += lax.dot_general(w3b, hx, (((0,), (0,)), ((), ())),
                             preferred_element_type=jnp.float32)
        out_ref[...] = jax.nn.sigmoid(r + b3_ref[...])


def _pool(aggp, degp, batch, W2, b2, W3, b3):
    out = pl.pallas_call(
        _pool_body,
        grid=(NB3,),
        in_specs=[
            pl.BlockSpec((B3, HID), lambda i: (i, 0)),
            pl.BlockSpec((B3, HID), lambda i: (i, 0)),
            pl.BlockSpec((B3,), lambda i: (i,)),
            pl.BlockSpec((B3,), lambda i: (i,)),
            pl.BlockSpec((B3,), lambda i: (i,)),
            pl.BlockSpec((HID, HID), lambda i: (0, 0)),
            pl.BlockSpec((1, HID), lambda i: (0, 0)),
            pl.BlockSpec((2 * HID, 1), lambda i: (0, 0)),
            pl.BlockSpec((1, 1), lambda i: (0, 0)),
        ],
        out_specs=pl.BlockSpec((1, G), lambda i: (0, 0)),
        out_shape=jax.ShapeDtypeStruct((1, G), jnp.float32),
        scratch_shapes=[
            pltpu.VMEM((HID, G), jnp.float32),
            pltpu.VMEM((HID, G), jnp.float32),
            pltpu.VMEM((1, G), jnp.float32),
        ],
    )(aggp[0], aggp[1], degp[0], degp[1], batch,
      W2, b2.reshape(1, HID), W3, b3.reshape(1, 1))
    return out.reshape(G)


def kernel(x, edge_index, batch, W1, b1, W2, b2, W3, b3):
    h = _compute_h(x, W1, b1)
    aggp, degp = _sc_aggregate(edge_index[0], edge_index[1], h)
    return _pool(aggp, degp, batch, W2, b2, W3, b3)
